# Initial kernel scaffold; baseline (speedup 1.0000x reference)
import jax
import jax.numpy as jnp
from jax.experimental import pallas as pl


def kernel(inputs, weight):
    raise NotImplementedError("write your pallas kernel here")


# speedup vs baseline: 2.7690x; 2.7690x over previous
import functools

import jax
import jax.numpy as jnp
from jax.experimental import pallas as pl
from jax.experimental.pallas import tpu as pltpu

_K = 8192
_D = 32
_N = 8192
_ROW_TILE = 256
_GRID = _N // _ROW_TILE
_COMMIT = 0.25


def _vq_body(x_ref, wt_ref, w_ref, idx_ref, enc_ref, qnt_ref, loss_ref,
             perp_ref, acc_ref, cnt_ref):
    step = pl.program_id(0)

    @pl.when(step == 0)
    def _init():
        acc_ref[...] = jnp.zeros_like(acc_ref)
        cnt_ref[...] = jnp.zeros_like(cnt_ref)

    x = x_ref[...]
    wt = wt_ref[...]
    idx = idx_ref[...]
    enc = (jax.lax.broadcasted_iota(jnp.int32, (_ROW_TILE, _K), 1)
           == idx).astype(jnp.float32)
    enc_ref[...] = enc
    prod = jax.lax.dot_general(x, wt, (((1,), (0,)), ((), ())),
                               preferred_element_type=jnp.float32,
                               precision=jax.lax.Precision.HIGHEST)
    wsq = jnp.sum(wt * wt, axis=0)
    scores = wsq[None, :] - 2.0 * prod
    sel = jnp.sum(scores * enc, axis=1)
    acc_ref[...] += jnp.sum(sel) + jnp.sum(x * x)
    qnt = jax.lax.dot_general(enc, w_ref[...], (((1,), (0,)), ((), ())),
                              preferred_element_type=jnp.float32,
                              precision=jax.lax.Precision.HIGHEST)
    qnt_ref[...] = qnt
    cnt_ref[...] += jnp.sum(enc, axis=0)[None, :]

    @pl.when(step == _GRID - 1)
    def _fin():
        loss = _COMMIT * acc_ref[0, 0] / (_N * _D)
        loss_ref[...] = jnp.reshape(loss, (1, 1))
        p = cnt_ref[...] * (1.0 / _N)
        ent = -jnp.sum(p * jnp.log(p + 1e-10))
        perp_ref[...] = jnp.reshape(jnp.exp(ent), (1, 1))


@jax.jit
def _vq_tc(flat, wt, w, idx):
    return pl.pallas_call(
        _vq_body,
        grid=(_GRID,),
        in_specs=[
            pl.BlockSpec((_ROW_TILE, _D), lambda i: (i, 0)),
            pl.BlockSpec((_D, _K), lambda i: (0, 0)),
            pl.BlockSpec((_K, _D), lambda i: (0, 0)),
            pl.BlockSpec((_ROW_TILE, 1), lambda i: (i, 0)),
        ],
        out_specs=[
            pl.BlockSpec((_ROW_TILE, _K), lambda i: (i, 0)),
            pl.BlockSpec((_ROW_TILE, _D), lambda i: (i, 0)),
            pl.BlockSpec((1, 1), lambda i: (0, 0)),
            pl.BlockSpec((1, 1), lambda i: (0, 0)),
        ],
        out_shape=[
            jax.ShapeDtypeStruct((_N, _K), jnp.float32),
            jax.ShapeDtypeStruct((_N, _D), jnp.float32),
            jax.ShapeDtypeStruct((1, 1), jnp.float32),
            jax.ShapeDtypeStruct((1, 1), jnp.float32),
        ],
        scratch_shapes=[
            pltpu.VMEM((1, 128), jnp.float32),
            pltpu.VMEM((1, _K), jnp.float32),
        ],
    )(flat, wt, w, idx)


def kernel(inputs, weight):
    xa = jnp.transpose(inputs, (0, 2, 3, 1)).reshape(_N, _D)
    distances = (jnp.sum(xa ** 2, axis=1, keepdims=True)
                 + jnp.sum(weight ** 2, axis=1)
                 - 2.0 * jnp.matmul(xa, weight.T))
    idx = jnp.argmin(distances, axis=1).astype(jnp.int32).reshape(_N, 1)

    inputs_p, weight_p = jax.lax.optimization_barrier((inputs, weight))
    x = jnp.transpose(inputs_p, (0, 2, 3, 1))
    flat = x.reshape(_N, _D)
    enc, qnt, loss, perp = _vq_tc(flat, weight_p.T, weight_p, idx)
    out_q = jnp.transpose(qnt.reshape(x.shape), (0, 3, 1, 2))
    return (loss[0, 0], out_q, perp[0, 0], enc)
